# initial kernel scaffold (unmeasured)
import jax
import jax.numpy as jnp
from jax import lax
from jax.experimental import pallas as pl
from jax.experimental.pallas import tpu as pltpu


def kernel(
    x,
):
    def body(*refs):
        pass

    out_shape = jax.ShapeDtypeStruct(..., jnp.float32)
    return pl.pallas_call(body, out_shape=out_shape)(...)


# baseline (device time: 315440 ns/iter reference)
import jax
import jax.numpy as jnp
from jax import lax
from jax.experimental import pallas as pl
from jax.experimental.pallas import tpu as pltpu

N_Y = 4

_H1_FROM_LEFT = 0
_H1_FROM_RIGHT = 1
_H2_FROM_LEFT = 2
_H2_FROM_RIGHT = 3


def kernel(x):
    m, n = x.shape
    half = m // 2
    M = N_Y * m

    def body(x_ref, out_ref, send_sems, recv_sems):
        my_x = lax.axis_index("x")
        my_y = lax.axis_index("y")
        my_z = lax.axis_index("z")
        left_y = (my_y - 1) % N_Y
        right_y = (my_y + 1) % N_Y
        opp_y = (my_y + 2) % N_Y
        left_id = (my_x, left_y, my_z)
        right_id = (my_x, right_y, my_z)

        barrier = pltpu.get_barrier_semaphore()
        for nbr in (left_id, right_id):
            pl.semaphore_signal(
                barrier, inc=1, device_id=nbr,
                device_id_type=pl.DeviceIdType.MESH,
            )
        pl.semaphore_wait(barrier, 2)

        out_ref[pl.ds(my_y * m, m), :] = x_ref[:, :].astype(out_ref.dtype)

        send_r1 = pltpu.make_async_remote_copy(
            src_ref=out_ref.at[pl.ds(my_y * m, m), :],
            dst_ref=out_ref.at[pl.ds(my_y * m, m), :],
            send_sem=send_sems.at[0],
            recv_sem=recv_sems.at[_H1_FROM_LEFT],
            device_id=right_id,
            device_id_type=pl.DeviceIdType.MESH,
        )
        send_l1 = pltpu.make_async_remote_copy(
            src_ref=out_ref.at[pl.ds(my_y * m, m), :],
            dst_ref=out_ref.at[pl.ds(my_y * m, m), :],
            send_sem=send_sems.at[1],
            recv_sem=recv_sems.at[_H1_FROM_RIGHT],
            device_id=left_id,
            device_id_type=pl.DeviceIdType.MESH,
        )
        send_r1.start()
        send_l1.start()

        recv_l1 = pltpu.make_async_remote_copy(
            src_ref=out_ref.at[pl.ds(left_y * m, m), :],
            dst_ref=out_ref.at[pl.ds(left_y * m, m), :],
            send_sem=send_sems.at[0],
            recv_sem=recv_sems.at[_H1_FROM_LEFT],
            device_id=left_id,
            device_id_type=pl.DeviceIdType.MESH,
        )
        recv_r1 = pltpu.make_async_remote_copy(
            src_ref=out_ref.at[pl.ds(right_y * m, m), :],
            dst_ref=out_ref.at[pl.ds(right_y * m, m), :],
            send_sem=send_sems.at[1],
            recv_sem=recv_sems.at[_H1_FROM_RIGHT],
            device_id=right_id,
            device_id_type=pl.DeviceIdType.MESH,
        )
        recv_l1.wait_recv()
        recv_r1.wait_recv()

        send_r2 = pltpu.make_async_remote_copy(
            src_ref=out_ref.at[pl.ds(left_y * m, half), :],
            dst_ref=out_ref.at[pl.ds(left_y * m, half), :],
            send_sem=send_sems.at[2],
            recv_sem=recv_sems.at[_H2_FROM_LEFT],
            device_id=right_id,
            device_id_type=pl.DeviceIdType.MESH,
        )
        send_l2 = pltpu.make_async_remote_copy(
            src_ref=out_ref.at[pl.ds(right_y * m + half, half), :],
            dst_ref=out_ref.at[pl.ds(right_y * m + half, half), :],
            send_sem=send_sems.at[3],
            recv_sem=recv_sems.at[_H2_FROM_RIGHT],
            device_id=left_id,
            device_id_type=pl.DeviceIdType.MESH,
        )
        send_r2.start()
        send_l2.start()

        recv_l2 = pltpu.make_async_remote_copy(
            src_ref=out_ref.at[pl.ds(opp_y * m, half), :],
            dst_ref=out_ref.at[pl.ds(opp_y * m, half), :],
            send_sem=send_sems.at[2],
            recv_sem=recv_sems.at[_H2_FROM_LEFT],
            device_id=left_id,
            device_id_type=pl.DeviceIdType.MESH,
        )
        recv_r2 = pltpu.make_async_remote_copy(
            src_ref=out_ref.at[pl.ds(opp_y * m + half, half), :],
            dst_ref=out_ref.at[pl.ds(opp_y * m + half, half), :],
            send_sem=send_sems.at[3],
            recv_sem=recv_sems.at[_H2_FROM_RIGHT],
            device_id=right_id,
            device_id_type=pl.DeviceIdType.MESH,
        )
        recv_l2.wait_recv()
        recv_r2.wait_recv()

        send_r1.wait_send()
        send_l1.wait_send()
        send_r2.wait_send()
        send_l2.wait_send()

    return pl.pallas_call(
        body,
        out_shape=jax.ShapeDtypeStruct((M, n), jnp.bfloat16),
        in_specs=[pl.BlockSpec(memory_space=pltpu.VMEM)],
        out_specs=pl.BlockSpec(memory_space=pltpu.VMEM),
        scratch_shapes=[
            pltpu.SemaphoreType.DMA((4,)),
            pltpu.SemaphoreType.DMA((4,)),
        ],
        compiler_params=pltpu.CompilerParams(collective_id=0),
    )(x)


# device time: 229643 ns/iter; 1.3736x vs baseline; 1.3736x over previous
import jax
import jax.numpy as jnp
from jax import lax
from jax.experimental import pallas as pl
from jax.experimental.pallas import tpu as pltpu

N_Y = 4
_MESH = pl.DeviceIdType.MESH


def kernel(x):
    m, n = x.shape
    half = m // 2
    M = N_Y * m

    def body(x_ref, out_ref, ysend_sems, yrecv_sems, xsend_sems, xrecv_sems):
        my_x = lax.axis_index("x")
        my_y = lax.axis_index("y")
        my_z = lax.axis_index("z")
        h_off = my_x * half
        o_off = half - h_off
        partner = (1 - my_x, my_y, my_z)

        def myslice(s):
            return out_ref.at[pl.ds(s * m + h_off, half), :]

        def othslice(s):
            return out_ref.at[pl.ds(s * m + o_off, half), :]

        barrier = pltpu.get_barrier_semaphore()
        for nbr in (
            (my_x, (my_y + 1) % N_Y, my_z),
            (my_x, (my_y - 1) % N_Y, my_z),
            partner,
        ):
            pl.semaphore_signal(
                barrier, inc=1, device_id=nbr, device_id_type=_MESH
            )
        pl.semaphore_wait(barrier, 3)

        out_ref[pl.ds(my_y * m, m), :] = x_ref[:, :].astype(out_ref.dtype)

        for Y in range(N_Y):

            @pl.when(my_y == Y)
            def _(Y=Y):
                sends = []

                def y_send(s, to_y, dirn):
                    c = pltpu.make_async_remote_copy(
                        src_ref=myslice(s),
                        dst_ref=myslice(s),
                        send_sem=ysend_sems.at[s, dirn],
                        recv_sem=yrecv_sems.at[s],
                        device_id=(my_x, to_y, my_z),
                        device_id_type=_MESH,
                    )
                    c.start()
                    sends.append(c)

                def x_send(s):
                    c = pltpu.make_async_remote_copy(
                        src_ref=myslice(s),
                        dst_ref=myslice(s),
                        send_sem=xsend_sems.at[s],
                        recv_sem=xrecv_sems.at[s],
                        device_id=partner,
                        device_id_type=_MESH,
                    )
                    c.start()
                    sends.append(c)

                def y_wait(s):
                    r = pltpu.make_async_remote_copy(
                        src_ref=myslice(s),
                        dst_ref=myslice(s),
                        send_sem=ysend_sems.at[s, 0],
                        recv_sem=yrecv_sems.at[s],
                        device_id=partner,
                        device_id_type=_MESH,
                    )
                    r.wait_recv()

                if Y < N_Y - 1:
                    y_send(Y, Y + 1, 1)
                if Y > 0:
                    y_send(Y, Y - 1, 0)

                for d in (1, 2, 3):
                    sl, sr = Y - d, Y + d
                    if sl >= 0:
                        y_wait(sl)
                        if Y < N_Y - 1:
                            y_send(sl, Y + 1, 1)
                        x_send(sl)
                    if sr <= N_Y - 1:
                        y_wait(sr)
                        if Y > 0:
                            y_send(sr, Y - 1, 0)
                        x_send(sr)

                for s in range(N_Y):
                    if s != Y:
                        r = pltpu.make_async_remote_copy(
                            src_ref=othslice(s),
                            dst_ref=othslice(s),
                            send_sem=xsend_sems.at[s],
                            recv_sem=xrecv_sems.at[s],
                            device_id=partner,
                            device_id_type=_MESH,
                        )
                        r.wait_recv()

                for c in sends:
                    c.wait_send()

    return pl.pallas_call(
        body,
        out_shape=jax.ShapeDtypeStruct((M, n), jnp.bfloat16),
        in_specs=[pl.BlockSpec(memory_space=pltpu.VMEM)],
        out_specs=pl.BlockSpec(memory_space=pltpu.VMEM),
        scratch_shapes=[
            pltpu.SemaphoreType.DMA((N_Y, 2)),
            pltpu.SemaphoreType.DMA((N_Y,)),
            pltpu.SemaphoreType.DMA((N_Y,)),
            pltpu.SemaphoreType.DMA((N_Y,)),
        ],
        compiler_params=pltpu.CompilerParams(collective_id=0),
    )(x)


# device time: 190228 ns/iter; 1.6582x vs baseline; 1.2072x over previous
import jax
import jax.numpy as jnp
from jax import lax
from jax.experimental import pallas as pl
from jax.experimental.pallas import tpu as pltpu

N_Y = 4
N_CHUNK = 8
_MESH = pl.DeviceIdType.MESH


def kernel(x):
    m, n = x.shape
    half = m // 2
    ch = half // N_CHUNK
    M = N_Y * m

    def body(x_ref, out_ref, ysend_sems, yrecv_sems, xsend_sems, xrecv_sems):
        my_x = lax.axis_index("x")
        my_y = lax.axis_index("y")
        my_z = lax.axis_index("z")
        h_off = my_x * half
        o_off = half - h_off
        partner = (1 - my_x, my_y, my_z)

        def mychunk(s, c):
            return out_ref.at[pl.ds(s * m + h_off + c * ch, ch), :]

        def othchunk(s, c):
            return out_ref.at[pl.ds(s * m + o_off + c * ch, ch), :]

        barrier = pltpu.get_barrier_semaphore()
        for nbr in (
            (my_x, (my_y + 1) % N_Y, my_z),
            (my_x, (my_y - 1) % N_Y, my_z),
            partner,
        ):
            pl.semaphore_signal(
                barrier, inc=1, device_id=nbr, device_id_type=_MESH
            )
        pl.semaphore_wait(barrier, 3)

        for Y in range(N_Y):

            @pl.when(my_y == Y)
            def _(Y=Y):
                sends = []

                def y_send(s, c, to_y, dirn):
                    cp = pltpu.make_async_remote_copy(
                        src_ref=mychunk(s, c),
                        dst_ref=mychunk(s, c),
                        send_sem=ysend_sems.at[s, dirn, c],
                        recv_sem=yrecv_sems.at[s, c],
                        device_id=(my_x, to_y, my_z),
                        device_id_type=_MESH,
                    )
                    cp.start()
                    sends.append(cp)

                def x_send(s, c):
                    cp = pltpu.make_async_remote_copy(
                        src_ref=mychunk(s, c),
                        dst_ref=mychunk(s, c),
                        send_sem=xsend_sems.at[s, c],
                        recv_sem=xrecv_sems.at[s, c],
                        device_id=partner,
                        device_id_type=_MESH,
                    )
                    cp.start()
                    sends.append(cp)

                def y_wait(s, c):
                    r = pltpu.make_async_remote_copy(
                        src_ref=mychunk(s, c),
                        dst_ref=mychunk(s, c),
                        send_sem=ysend_sems.at[s, 0, c],
                        recv_sem=yrecv_sems.at[s, c],
                        device_id=partner,
                        device_id_type=_MESH,
                    )
                    r.wait_recv()

                for c in range(N_CHUNK):
                    row = h_off + c * ch
                    out_ref[pl.ds(Y * m + row, ch), :] = x_ref[
                        pl.ds(row, ch), :
                    ].astype(out_ref.dtype)
                    if Y < N_Y - 1:
                        y_send(Y, c, Y + 1, 1)
                    if Y > 0:
                        y_send(Y, c, Y - 1, 0)

                out_ref[pl.ds(Y * m + o_off, half), :] = x_ref[
                    pl.ds(o_off, half), :
                ].astype(out_ref.dtype)

                for d in (1, 2, 3):
                    sl, sr = Y - d, Y + d
                    if sl >= 0:
                        for c in range(N_CHUNK):
                            y_wait(sl, c)
                            if Y < N_Y - 1:
                                y_send(sl, c, Y + 1, 1)
                            x_send(sl, c)
                    if sr <= N_Y - 1:
                        for c in range(N_CHUNK):
                            y_wait(sr, c)
                            if Y > 0:
                                y_send(sr, c, Y - 1, 0)
                            x_send(sr, c)

                for s in range(N_Y):
                    if s != Y:
                        for c in range(N_CHUNK):
                            r = pltpu.make_async_remote_copy(
                                src_ref=othchunk(s, c),
                                dst_ref=othchunk(s, c),
                                send_sem=xsend_sems.at[s, c],
                                recv_sem=xrecv_sems.at[s, c],
                                device_id=partner,
                                device_id_type=_MESH,
                            )
                            r.wait_recv()

                for cp in sends:
                    cp.wait_send()

    return pl.pallas_call(
        body,
        out_shape=jax.ShapeDtypeStruct((M, n), jnp.bfloat16),
        in_specs=[pl.BlockSpec(memory_space=pltpu.VMEM)],
        out_specs=pl.BlockSpec(memory_space=pltpu.VMEM),
        scratch_shapes=[
            pltpu.SemaphoreType.DMA((N_Y, 2, N_CHUNK)),
            pltpu.SemaphoreType.DMA((N_Y, N_CHUNK)),
            pltpu.SemaphoreType.DMA((N_Y, N_CHUNK)),
            pltpu.SemaphoreType.DMA((N_Y, N_CHUNK)),
        ],
        compiler_params=pltpu.CompilerParams(collective_id=0),
    )(x)


# device time: 179429 ns/iter; 1.7580x vs baseline; 1.0602x over previous
import jax
import jax.numpy as jnp
from jax import lax
from jax.experimental import pallas as pl
from jax.experimental.pallas import tpu as pltpu

N_Y = 4
N_CHUNK = 8
N_STORE = 8
_MESH = pl.DeviceIdType.MESH


def kernel(x):
    m, n = x.shape
    half = m // 2
    ch = half // N_CHUNK
    M = N_Y * m

    def body(
        x_hbm,
        out_hbm,
        comm,
        stage,
        ysend_sems,
        yrecv_sems,
        xsend_sems,
        xrecv_sems,
        load_sems,
        store_sems,
    ):
        my_x = lax.axis_index("x")
        my_y = lax.axis_index("y")
        my_z = lax.axis_index("z")
        h_off = my_x * half
        o_off = half - h_off
        partner = (1 - my_x, my_y, my_z)

        def mychunk(s, c):
            return comm.at[pl.ds(s * m + h_off + c * ch, ch), :]

        def othchunk(s, c):
            return comm.at[pl.ds(s * m + o_off + c * ch, ch), :]

        barrier = pltpu.get_barrier_semaphore()
        for nbr in (
            (my_x, (my_y + 1) % N_Y, my_z),
            (my_x, (my_y - 1) % N_Y, my_z),
            partner,
        ):
            pl.semaphore_signal(
                barrier, inc=1, device_id=nbr, device_id_type=_MESH
            )
        pl.semaphore_wait(barrier, 3)

        for Y in range(N_Y):

            @pl.when(my_y == Y)
            def _(Y=Y):
                sends = []
                stores = []

                def y_send(s, c, to_y, dirn):
                    cp = pltpu.make_async_remote_copy(
                        src_ref=mychunk(s, c),
                        dst_ref=mychunk(s, c),
                        send_sem=ysend_sems.at[s, dirn, c],
                        recv_sem=yrecv_sems.at[s, c],
                        device_id=(my_x, to_y, my_z),
                        device_id_type=_MESH,
                    )
                    cp.start()
                    sends.append(cp)

                def x_send(s, c):
                    cp = pltpu.make_async_remote_copy(
                        src_ref=mychunk(s, c),
                        dst_ref=mychunk(s, c),
                        send_sem=xsend_sems.at[s, c],
                        recv_sem=xrecv_sems.at[s, c],
                        device_id=partner,
                        device_id_type=_MESH,
                    )
                    cp.start()
                    sends.append(cp)

                def y_wait(s, c):
                    r = pltpu.make_async_remote_copy(
                        src_ref=mychunk(s, c),
                        dst_ref=mychunk(s, c),
                        send_sem=ysend_sems.at[s, 0, c],
                        recv_sem=yrecv_sems.at[s, c],
                        device_id=partner,
                        device_id_type=_MESH,
                    )
                    r.wait_recv()

                def store_out(row):
                    if len(stores) >= N_STORE:
                        stores[len(stores) - N_STORE].wait()
                    cp = pltpu.make_async_copy(
                        comm.at[pl.ds(row, ch), :],
                        out_hbm.at[pl.ds(row, ch), :],
                        store_sems.at[len(stores) % N_STORE],
                    )
                    cp.start()
                    stores.append(cp)

                n_own = 2 * N_CHUNK
                def own_row(k):
                    if k < N_CHUNK:
                        return h_off + k * ch
                    return o_off + (k - N_CHUNK) * ch

                loads = []
                for k in range(min(2, n_own)):
                    cp = pltpu.make_async_copy(
                        x_hbm.at[pl.ds(own_row(k), ch), :],
                        stage.at[k % 2],
                        load_sems.at[k % 2],
                    )
                    cp.start()
                    loads.append(cp)
                for k in range(n_own):
                    loads[k].wait()
                    row = own_row(k)
                    comm[pl.ds(Y * m + row, ch), :] = stage[k % 2].astype(
                        comm.dtype
                    )
                    if k + 2 < n_own:
                        cp = pltpu.make_async_copy(
                            x_hbm.at[pl.ds(own_row(k + 2), ch), :],
                            stage.at[k % 2],
                            load_sems.at[k % 2],
                        )
                        cp.start()
                        loads.append(cp)
                    if k < N_CHUNK:
                        if Y < N_Y - 1:
                            y_send(Y, k, Y + 1, 1)
                        if Y > 0:
                            y_send(Y, k, Y - 1, 0)
                    store_out(Y * m + row)

                for d in (1, 2, 3):
                    sl, sr = Y - d, Y + d
                    if sl >= 0:
                        for c in range(N_CHUNK):
                            y_wait(sl, c)
                            if Y < N_Y - 1:
                                y_send(sl, c, Y + 1, 1)
                            x_send(sl, c)
                            store_out(sl * m + h_off + c * ch)
                    if sr <= N_Y - 1:
                        for c in range(N_CHUNK):
                            y_wait(sr, c)
                            if Y > 0:
                                y_send(sr, c, Y - 1, 0)
                            x_send(sr, c)
                            store_out(sr * m + h_off + c * ch)

                for s in range(N_Y):
                    if s != Y:
                        for c in range(N_CHUNK):
                            r = pltpu.make_async_remote_copy(
                                src_ref=othchunk(s, c),
                                dst_ref=othchunk(s, c),
                                send_sem=xsend_sems.at[s, c],
                                recv_sem=xrecv_sems.at[s, c],
                                device_id=partner,
                                device_id_type=_MESH,
                            )
                            r.wait_recv()
                            store_out(s * m + o_off + c * ch)

                for cp in sends:
                    cp.wait_send()
                for cp in stores[max(0, len(stores) - N_STORE):]:
                    cp.wait()

    return pl.pallas_call(
        body,
        out_shape=jax.ShapeDtypeStruct((M, n), jnp.bfloat16),
        in_specs=[pl.BlockSpec(memory_space=pl.ANY)],
        out_specs=pl.BlockSpec(memory_space=pl.ANY),
        scratch_shapes=[
            pltpu.VMEM((M, n), jnp.bfloat16),
            pltpu.VMEM((2, ch, n), jnp.float32),
            pltpu.SemaphoreType.DMA((N_Y, 2, N_CHUNK)),
            pltpu.SemaphoreType.DMA((N_Y, N_CHUNK)),
            pltpu.SemaphoreType.DMA((N_Y, N_CHUNK)),
            pltpu.SemaphoreType.DMA((N_Y, N_CHUNK)),
            pltpu.SemaphoreType.DMA((2,)),
            pltpu.SemaphoreType.DMA((N_STORE,)),
        ],
        compiler_params=pltpu.CompilerParams(
            collective_id=0, vmem_limit_bytes=48 * 1024 * 1024
        ),
    )(x)


# device time: 177283 ns/iter; 1.7793x vs baseline; 1.0121x over previous
import jax
import jax.numpy as jnp
from jax import lax
from jax.experimental import pallas as pl
from jax.experimental.pallas import tpu as pltpu

N_Y = 4
N_CHUNK = 8
N_STORE = 8
_MESH = pl.DeviceIdType.MESH


def kernel(x):
    m, n = x.shape
    half = m // 2
    ch = half // N_CHUNK
    M = N_Y * m

    def body(
        x_hbm,
        out_hbm,
        comm,
        stage,
        ysend_sems,
        yrecv_sems,
        xsend_sems,
        xrecv_sems,
        load_sems,
        store_sems,
    ):
        my_x = lax.axis_index("x")
        my_y = lax.axis_index("y")
        my_z = lax.axis_index("z")
        h_off = my_x * half
        o_off = half - h_off
        partner = (1 - my_x, my_y, my_z)

        def mychunk(s, c):
            return comm.at[pl.ds(s * m + h_off + c * ch, ch), :]

        def othchunk(s, c):
            return comm.at[pl.ds(s * m + o_off + c * ch, ch), :]

        barrier = pltpu.get_barrier_semaphore()

        for Y in range(N_Y):

            @pl.when(my_y == Y)
            def _(Y=Y):
                nbrs = [partner]
                if Y > 0:
                    nbrs.append((my_x, Y - 1, my_z))
                if Y < N_Y - 1:
                    nbrs.append((my_x, Y + 1, my_z))
                for nbr in nbrs:
                    pl.semaphore_signal(
                        barrier, inc=1, device_id=nbr, device_id_type=_MESH
                    )
                pl.semaphore_wait(barrier, len(nbrs))

                sends = []
                stores = []

                def y_send(s, c, to_y, dirn):
                    cp = pltpu.make_async_remote_copy(
                        src_ref=mychunk(s, c),
                        dst_ref=mychunk(s, c),
                        send_sem=ysend_sems.at[s, dirn, c],
                        recv_sem=yrecv_sems.at[s, c],
                        device_id=(my_x, to_y, my_z),
                        device_id_type=_MESH,
                    )
                    cp.start()
                    sends.append(cp)

                def x_send(s, c):
                    cp = pltpu.make_async_remote_copy(
                        src_ref=mychunk(s, c),
                        dst_ref=mychunk(s, c),
                        send_sem=xsend_sems.at[s, c],
                        recv_sem=xrecv_sems.at[s, c],
                        device_id=partner,
                        device_id_type=_MESH,
                    )
                    cp.start()
                    sends.append(cp)

                def y_wait(s, c):
                    r = pltpu.make_async_remote_copy(
                        src_ref=mychunk(s, c),
                        dst_ref=mychunk(s, c),
                        send_sem=ysend_sems.at[s, 0, c],
                        recv_sem=yrecv_sems.at[s, c],
                        device_id=partner,
                        device_id_type=_MESH,
                    )
                    r.wait_recv()

                def store_out(row):
                    if len(stores) >= N_STORE:
                        stores[len(stores) - N_STORE].wait()
                    cp = pltpu.make_async_copy(
                        comm.at[pl.ds(row, ch), :],
                        out_hbm.at[pl.ds(row, ch), :],
                        store_sems.at[len(stores) % N_STORE],
                    )
                    cp.start()
                    stores.append(cp)

                n_own = 2 * N_CHUNK
                def own_row(k):
                    if k < N_CHUNK:
                        return h_off + k * ch
                    return o_off + (k - N_CHUNK) * ch

                loads = []
                for k in range(min(2, n_own)):
                    cp = pltpu.make_async_copy(
                        x_hbm.at[pl.ds(own_row(k), ch), :],
                        stage.at[k % 2],
                        load_sems.at[k % 2],
                    )
                    cp.start()
                    loads.append(cp)
                for k in range(n_own):
                    loads[k].wait()
                    row = own_row(k)
                    comm[pl.ds(Y * m + row, ch), :] = stage[k % 2].astype(
                        comm.dtype
                    )
                    if k + 2 < n_own:
                        cp = pltpu.make_async_copy(
                            x_hbm.at[pl.ds(own_row(k + 2), ch), :],
                            stage.at[k % 2],
                            load_sems.at[k % 2],
                        )
                        cp.start()
                        loads.append(cp)
                    if k < N_CHUNK:
                        if Y < N_Y - 1:
                            y_send(Y, k, Y + 1, 1)
                        if Y > 0:
                            y_send(Y, k, Y - 1, 0)
                    store_out(Y * m + row)

                for d in (1, 2, 3):
                    sl, sr = Y - d, Y + d
                    if sl >= 0:
                        for c in range(N_CHUNK):
                            y_wait(sl, c)
                            if Y < N_Y - 1:
                                y_send(sl, c, Y + 1, 1)
                            x_send(sl, c)
                            store_out(sl * m + h_off + c * ch)
                    if sr <= N_Y - 1:
                        for c in range(N_CHUNK):
                            y_wait(sr, c)
                            if Y > 0:
                                y_send(sr, c, Y - 1, 0)
                            x_send(sr, c)
                            store_out(sr * m + h_off + c * ch)

                for s in range(N_Y):
                    if s != Y:
                        for c in range(N_CHUNK):
                            r = pltpu.make_async_remote_copy(
                                src_ref=othchunk(s, c),
                                dst_ref=othchunk(s, c),
                                send_sem=xsend_sems.at[s, c],
                                recv_sem=xrecv_sems.at[s, c],
                                device_id=partner,
                                device_id_type=_MESH,
                            )
                            r.wait_recv()
                            store_out(s * m + o_off + c * ch)

                for cp in sends:
                    cp.wait_send()
                for cp in stores[max(0, len(stores) - N_STORE):]:
                    cp.wait()

    out = pl.pallas_call(
        body,
        out_shape=jax.ShapeDtypeStruct((M, n), jnp.bfloat16),
        in_specs=[pl.BlockSpec(memory_space=pl.ANY)],
        out_specs=pl.BlockSpec(memory_space=pl.ANY),
        scratch_shapes=[
            pltpu.VMEM((M, n), jnp.bfloat16),
            pltpu.VMEM((2, ch, n), jnp.float32),
            pltpu.SemaphoreType.DMA((N_Y, 2, N_CHUNK)),
            pltpu.SemaphoreType.DMA((N_Y, N_CHUNK)),
            pltpu.SemaphoreType.DMA((N_Y, N_CHUNK)),
            pltpu.SemaphoreType.DMA((N_Y, N_CHUNK)),
            pltpu.SemaphoreType.DMA((2,)),
            pltpu.SemaphoreType.DMA((N_STORE,)),
        ],
        compiler_params=pltpu.CompilerParams(
            collective_id=1, vmem_limit_bytes=48 * 1024 * 1024
        ),
    )(x)
    return lax.optimization_barrier(out)
